# baseline (speedup 1.0000x reference)
import jax
import jax.numpy as jnp
from jax.experimental import pallas as pl
from jax.experimental.pallas import tpu as pltpu

N = 100000
D = 128
K = 16
OUT = 256
S = 8
N8 = N // S
TNB = 1280
NTILES = (N8 + TNB - 1) // TNB


def _pass1(x_ref, evq_ref, evals_ref, mask_ref, w1t_ref, b1_ref, w2t_ref,
           b2_ref, wp_ref, m_ref, acc_ref):
    i = pl.program_id(0)

    @pl.when(i == 0)
    def _():
        acc_ref[...] = jnp.zeros_like(acc_ref)

    row = jax.lax.broadcasted_iota(jnp.int32, (TNB, 1), 0) + i * TNB
    valid = row < N8
    evq = jnp.where(valid, evq_ref[...], 0.0)
    p = jnp.zeros((K, D), dtype=jnp.float32)
    for s in range(S):
        xs = jnp.where(valid, x_ref[s], 0.0)
        p += jax.lax.dot_general(
            evq[:, 16 * s:16 * (s + 1)], xs,
            dimension_numbers=(((0,), (0,)), ((), ())),
            preferred_element_type=jnp.float32)
    acc_ref[...] += p

    @pl.when(i == pl.num_programs(0) - 1)
    def _():
        h = jnp.maximum(
            jnp.dot(w1t_ref[...], evals_ref[...],
                    preferred_element_type=jnp.float32) + b1_ref[...], 0.0)
        filt = jnp.tanh(
            jnp.dot(w2t_ref[...], h,
                    preferred_element_type=jnp.float32) + b2_ref[...])
        filt = filt * mask_ref[...]
        m_ref[...] = jnp.dot(filt * acc_ref[...], wp_ref[...],
                             preferred_element_type=jnp.float32)


def _pass2(evq_ref, m_ref, bp_ref, out_ref):
    evq = evq_ref[...]
    for s in range(S):
        out_ref[s] = jnp.dot(evq[:, 16 * s:16 * (s + 1)], m_ref[...],
                             preferred_element_type=jnp.float32) + bp_ref[...]


def kernel(x, eigvecs, eigvals, eig_mask, W1, b1, W2, b2, Wp, bp):
    evq = eigvecs.reshape(S, N8, K).transpose(1, 0, 2).reshape(N8, S * K)
    x3 = x.reshape(S, N8, D)
    evals_col = eigvals.reshape(K, 1)
    mask_col = eig_mask.astype(jnp.float32).reshape(K, 1)
    w1t = W1.T
    b1_col = b1.reshape(K // 2, 1)
    w2t = W2.T
    b2_col = b2.reshape(K, 1)
    bp_row = bp.reshape(1, OUT)

    m = pl.pallas_call(
        _pass1,
        grid=(NTILES,),
        in_specs=[
            pl.BlockSpec((S, TNB, D), lambda i: (0, i, 0)),
            pl.BlockSpec((TNB, S * K), lambda i: (i, 0)),
            pl.BlockSpec((K, 1), lambda i: (0, 0)),
            pl.BlockSpec((K, 1), lambda i: (0, 0)),
            pl.BlockSpec((K // 2, K), lambda i: (0, 0)),
            pl.BlockSpec((K // 2, 1), lambda i: (0, 0)),
            pl.BlockSpec((K, K // 2), lambda i: (0, 0)),
            pl.BlockSpec((K, 1), lambda i: (0, 0)),
            pl.BlockSpec((D, OUT), lambda i: (0, 0)),
        ],
        out_specs=pl.BlockSpec((K, OUT), lambda i: (0, 0)),
        out_shape=jax.ShapeDtypeStruct((K, OUT), jnp.float32),
        scratch_shapes=[pltpu.VMEM((K, D), jnp.float32)],
    )(x3, evq, evals_col, mask_col, w1t, b1_col, w2t, b2_col, Wp)

    out3 = pl.pallas_call(
        _pass2,
        grid=(NTILES,),
        in_specs=[
            pl.BlockSpec((TNB, S * K), lambda i: (i, 0)),
            pl.BlockSpec((K, OUT), lambda i: (0, 0)),
            pl.BlockSpec((1, OUT), lambda i: (0, 0)),
        ],
        out_specs=pl.BlockSpec((S, TNB, OUT), lambda i: (0, i, 0)),
        out_shape=jax.ShapeDtypeStruct((S, N8, OUT), jnp.float32),
    )(evq, m, bp_row)
    return out3.reshape(N, OUT)


# speedup vs baseline: 1.3848x; 1.3848x over previous
import jax
import jax.numpy as jnp
from jax.experimental import pallas as pl
from jax.experimental.pallas import tpu as pltpu

N = 100000
D = 128
K = 16
OUT = 256
F = 8
N8 = N // F
TNB = 500
NTILES = N8 // TNB


def _pass1(x_ref, evp_ref, evals_ref, mask_ref, w1t_ref, b1_ref, w2t_ref,
           b2_ref, wp_ref, m_ref, acc_ref):
    i = pl.program_id(0)

    @pl.when(i == 0)
    def _():
        acc_ref[...] = jnp.zeros_like(acc_ref)

    evp = evp_ref[:, 0, :]
    p = jnp.zeros((K, D), dtype=jnp.float32)
    for s in range(F):
        p += jax.lax.dot_general(
            evp[:, K * s:K * (s + 1)], x_ref[:, s, :],
            dimension_numbers=(((0,), (0,)), ((), ())),
            preferred_element_type=jnp.float32)
    acc_ref[...] += p

    @pl.when(i == pl.num_programs(0) - 1)
    def _():
        h = jnp.maximum(
            jnp.dot(w1t_ref[...], evals_ref[...],
                    preferred_element_type=jnp.float32) + b1_ref[...], 0.0)
        filt = jnp.tanh(
            jnp.dot(w2t_ref[...], h,
                    preferred_element_type=jnp.float32) + b2_ref[...])
        filt = filt * mask_ref[...]
        m_ref[...] = jnp.dot(filt * acc_ref[...], wp_ref[...],
                             preferred_element_type=jnp.float32)


def _pass2(evp_ref, m_ref, bp_ref, out_ref):
    evp = evp_ref[:, 0, :]
    for s in range(F):
        out_ref[:, s, :] = jnp.dot(
            evp[:, K * s:K * (s + 1)], m_ref[...],
            preferred_element_type=jnp.float32) + bp_ref[...]


def kernel(x, eigvecs, eigvals, eig_mask, W1, b1, W2, b2, Wp, bp):
    evp = eigvecs.reshape(N8, 1, F * K)
    x4 = x.reshape(N8, F, D)
    evals_col = eigvals.reshape(K, 1)
    mask_col = eig_mask.astype(jnp.float32).reshape(K, 1)
    w1t = W1.T
    b1_col = b1.reshape(K // 2, 1)
    w2t = W2.T
    b2_col = b2.reshape(K, 1)
    bp_row = bp.reshape(1, OUT)

    m = pl.pallas_call(
        _pass1,
        grid=(NTILES,),
        in_specs=[
            pl.BlockSpec((TNB, F, D), lambda i: (i, 0, 0)),
            pl.BlockSpec((TNB, 1, F * K), lambda i: (i, 0, 0)),
            pl.BlockSpec((K, 1), lambda i: (0, 0)),
            pl.BlockSpec((K, 1), lambda i: (0, 0)),
            pl.BlockSpec((K // 2, K), lambda i: (0, 0)),
            pl.BlockSpec((K // 2, 1), lambda i: (0, 0)),
            pl.BlockSpec((K, K // 2), lambda i: (0, 0)),
            pl.BlockSpec((K, 1), lambda i: (0, 0)),
            pl.BlockSpec((D, OUT), lambda i: (0, 0)),
        ],
        out_specs=pl.BlockSpec((K, OUT), lambda i: (0, 0)),
        out_shape=jax.ShapeDtypeStruct((K, OUT), jnp.float32),
        scratch_shapes=[pltpu.VMEM((K, D), jnp.float32)],
    )(x4, evp, evals_col, mask_col, w1t, b1_col, w2t, b2_col, Wp)

    out4 = pl.pallas_call(
        _pass2,
        grid=(NTILES,),
        in_specs=[
            pl.BlockSpec((TNB, 1, F * K), lambda i: (i, 0, 0)),
            pl.BlockSpec((K, OUT), lambda i: (0, 0)),
            pl.BlockSpec((1, OUT), lambda i: (0, 0)),
        ],
        out_specs=pl.BlockSpec((TNB, F, OUT), lambda i: (i, 0, 0)),
        out_shape=jax.ShapeDtypeStruct((N8, F, OUT), jnp.float32),
    )(evp, m, bp_row)
    return out4.reshape(N, OUT)


# speedup vs baseline: 3.1821x; 2.2979x over previous
import jax
import jax.numpy as jnp
from jax.experimental import pallas as pl
from jax.experimental.pallas import tpu as pltpu

N = 100000
D = 128
K = 16
OUT = 256
TN = 6400
NP = 102400
NT = NP // TN


def _body(x_ref, evt_ref, evals_ref, mask_ref, w1t_ref, b1_ref, w2t_ref,
          b2_ref, wp_ref, bp_ref, out_ref, acc_ref, m_ref):
    p = pl.program_id(0)
    j = pl.program_id(1)
    evt = evt_ref[:, pl.ds(j * TN, TN)]

    @pl.when(jnp.logical_and(p == 0, j == 0))
    def _():
        acc_ref[...] = jnp.zeros_like(acc_ref)

    @pl.when(p == 0)
    def _():
        row = jax.lax.broadcasted_iota(jnp.int32, (TN, 1), 0) + j * TN
        xm = jnp.where(row < N, x_ref[...], 0.0)
        acc_ref[...] += jax.lax.dot_general(
            evt, xm,
            dimension_numbers=(((1,), (0,)), ((), ())),
            preferred_element_type=jnp.float32)

    @pl.when(jnp.logical_and(p == 1, j == 0))
    def _():
        h = jnp.maximum(
            jnp.dot(w1t_ref[...], evals_ref[...],
                    preferred_element_type=jnp.float32) + b1_ref[...], 0.0)
        filt = jnp.tanh(
            jnp.dot(w2t_ref[...], h,
                    preferred_element_type=jnp.float32) + b2_ref[...])
        filt = filt * mask_ref[...]
        m_ref[...] = jnp.dot(filt * acc_ref[...], wp_ref[...],
                             preferred_element_type=jnp.float32)

    @pl.when(p == 1)
    def _():
        out_ref[...] = jax.lax.dot_general(
            evt, m_ref[...],
            dimension_numbers=(((0,), (0,)), ((), ())),
            preferred_element_type=jnp.float32) + bp_ref[...]


def kernel(x, eigvecs, eigvals, eig_mask, W1, b1, W2, b2, Wp, bp):
    evt = jnp.pad(eigvecs.T, ((0, 0), (0, NP - N)))
    evals_col = eigvals.reshape(K, 1)
    mask_col = eig_mask.astype(jnp.float32).reshape(K, 1)
    w1t = W1.T
    b1_col = b1.reshape(K // 2, 1)
    w2t = W2.T
    b2_col = b2.reshape(K, 1)
    bp_row = bp.reshape(1, OUT)

    out = pl.pallas_call(
        _body,
        grid=(2, NT),
        in_specs=[
            pl.BlockSpec((TN, D), lambda p, j: ((1 - p) * j + p * (NT - 1), 0)),
            pl.BlockSpec((K, NP), lambda p, j: (0, 0)),
            pl.BlockSpec((K, 1), lambda p, j: (0, 0)),
            pl.BlockSpec((K, 1), lambda p, j: (0, 0)),
            pl.BlockSpec((K // 2, K), lambda p, j: (0, 0)),
            pl.BlockSpec((K // 2, 1), lambda p, j: (0, 0)),
            pl.BlockSpec((K, K // 2), lambda p, j: (0, 0)),
            pl.BlockSpec((K, 1), lambda p, j: (0, 0)),
            pl.BlockSpec((D, OUT), lambda p, j: (0, 0)),
            pl.BlockSpec((1, OUT), lambda p, j: (0, 0)),
        ],
        out_specs=pl.BlockSpec((TN, OUT), lambda p, j: (p * j, 0)),
        out_shape=jax.ShapeDtypeStruct((N, OUT), jnp.float32),
        scratch_shapes=[pltpu.VMEM((K, D), jnp.float32),
                        pltpu.VMEM((K, OUT), jnp.float32)],
    )(x, evt, evals_col, mask_col, w1t, b1_col, w2t, b2_col, Wp, bp_row)
    return out


# speedup vs baseline: 3.2303x; 1.0151x over previous
import jax
import jax.numpy as jnp
from jax.experimental import pallas as pl
from jax.experimental.pallas import tpu as pltpu

N = 100000
D = 128
K = 16
OUT = 256
TN = 6400
NP = 102400
NT = NP // TN


def _body(x_ref, evt_ref, evals_ref, mask_ref, w1t_ref, b1_ref, w2t_ref,
          b2_ref, wp_ref, bp_ref, out_ref, acc_ref, m_ref):
    p = pl.program_id(0)
    j = pl.program_id(1)
    evt = evt_ref[:, pl.ds(j * TN, TN)]

    @pl.when(jnp.logical_and(p == 0, j == 0))
    def _():
        acc_ref[...] = jnp.zeros_like(acc_ref)

    @pl.when(jnp.logical_and(p == 0, j < NT - 1))
    def _():
        acc_ref[...] += jax.lax.dot_general(
            evt, x_ref[...],
            dimension_numbers=(((1,), (0,)), ((), ())),
            preferred_element_type=jnp.float32)

    @pl.when(jnp.logical_and(p == 0, j == NT - 1))
    def _():
        row = jax.lax.broadcasted_iota(jnp.int32, (TN, 1), 0) + j * TN
        xm = jnp.where(row < N, x_ref[...], 0.0)
        acc_ref[...] += jax.lax.dot_general(
            evt, xm,
            dimension_numbers=(((1,), (0,)), ((), ())),
            preferred_element_type=jnp.float32)

    @pl.when(jnp.logical_and(p == 1, j == 0))
    def _():
        h = jnp.maximum(
            jnp.dot(w1t_ref[...], evals_ref[...],
                    preferred_element_type=jnp.float32) + b1_ref[...], 0.0)
        filt = jnp.tanh(
            jnp.dot(w2t_ref[...], h,
                    preferred_element_type=jnp.float32) + b2_ref[...])
        filt = filt * mask_ref[...]
        m_ref[...] = jnp.dot(filt * acc_ref[...], wp_ref[...],
                             preferred_element_type=jnp.float32)

    @pl.when(p == 1)
    def _():
        out_ref[...] = jax.lax.dot_general(
            evt, m_ref[...],
            dimension_numbers=(((0,), (0,)), ((), ())),
            preferred_element_type=jnp.float32) + bp_ref[...]


def kernel(x, eigvecs, eigvals, eig_mask, W1, b1, W2, b2, Wp, bp):
    evt = jnp.pad(eigvecs.T, ((0, 0), (0, NP - N)))
    evals_col = eigvals.reshape(K, 1)
    mask_col = eig_mask.astype(jnp.float32).reshape(K, 1)
    w1t = W1.T
    b1_col = b1.reshape(K // 2, 1)
    w2t = W2.T
    b2_col = b2.reshape(K, 1)
    bp_row = bp.reshape(1, OUT)

    out = pl.pallas_call(
        _body,
        grid=(2, NT),
        in_specs=[
            pl.BlockSpec((TN, D), lambda p, j: ((1 - p) * j + p * (NT - 1), 0)),
            pl.BlockSpec((K, NP), lambda p, j: (0, 0)),
            pl.BlockSpec((K, 1), lambda p, j: (0, 0)),
            pl.BlockSpec((K, 1), lambda p, j: (0, 0)),
            pl.BlockSpec((K // 2, K), lambda p, j: (0, 0)),
            pl.BlockSpec((K // 2, 1), lambda p, j: (0, 0)),
            pl.BlockSpec((K, K // 2), lambda p, j: (0, 0)),
            pl.BlockSpec((K, 1), lambda p, j: (0, 0)),
            pl.BlockSpec((D, OUT), lambda p, j: (0, 0)),
            pl.BlockSpec((1, OUT), lambda p, j: (0, 0)),
        ],
        out_specs=pl.BlockSpec((TN, OUT), lambda p, j: (p * j, 0)),
        out_shape=jax.ShapeDtypeStruct((N, OUT), jnp.float32),
        scratch_shapes=[pltpu.VMEM((K, D), jnp.float32),
                        pltpu.VMEM((K, OUT), jnp.float32)],
    )(x, evt, evals_col, mask_col, w1t, b1_col, w2t, b2_col, Wp, bp_row)
    return out


# speedup vs baseline: 3.5463x; 1.0978x over previous
import jax
import jax.numpy as jnp
from jax.experimental import pallas as pl
from jax.experimental.pallas import tpu as pltpu

N = 100000
D = 128
K = 16
OUT = 256
TN = 6400
NT = (N + TN - 1) // TN
TAIL = N - (NT - 1) * TN


def _body(x_ref, evt_ref, evals_ref, mask_ref, w1t_ref, b1_ref, w2t_ref,
          b2_ref, wp_ref, bp_ref, out_ref, acc_ref, m_ref):
    p = pl.program_id(0)
    j = pl.program_id(1)

    @pl.when(jnp.logical_and(p == 0, j == 0))
    def _():
        acc_ref[...] = jnp.zeros_like(acc_ref)

    @pl.when(jnp.logical_and(p == 0, j < NT - 1))
    def _():
        evt = evt_ref[:, pl.ds(j * TN, TN)]
        acc_ref[...] += jax.lax.dot_general(
            evt, x_ref[...],
            dimension_numbers=(((1,), (0,)), ((), ())),
            preferred_element_type=jnp.float32)

    @pl.when(jnp.logical_and(p == 0, j == NT - 1))
    def _():
        evt = evt_ref[:, pl.ds((NT - 1) * TN, TAIL)]
        acc_ref[...] += jax.lax.dot_general(
            evt, x_ref[0:TAIL, :],
            dimension_numbers=(((1,), (0,)), ((), ())),
            preferred_element_type=jnp.float32)

    @pl.when(jnp.logical_and(p == 1, j == 0))
    def _():
        h = jnp.maximum(
            jnp.dot(w1t_ref[...], evals_ref[...],
                    preferred_element_type=jnp.float32) + b1_ref[...], 0.0)
        filt = jnp.tanh(
            jnp.dot(w2t_ref[...], h,
                    preferred_element_type=jnp.float32) + b2_ref[...])
        filt = filt * mask_ref[...]
        m_ref[...] = jnp.dot(filt * acc_ref[...], wp_ref[...],
                             preferred_element_type=jnp.float32)

    @pl.when(jnp.logical_and(p == 1, j < NT - 1))
    def _():
        evt = evt_ref[:, pl.ds(j * TN, TN)]
        out_ref[...] = jax.lax.dot_general(
            evt, m_ref[...],
            dimension_numbers=(((0,), (0,)), ((), ())),
            preferred_element_type=jnp.float32) + bp_ref[...]

    @pl.when(jnp.logical_and(p == 1, j == NT - 1))
    def _():
        evt = evt_ref[:, pl.ds((NT - 1) * TN, TAIL)]
        out_ref[0:TAIL, :] = jax.lax.dot_general(
            evt, m_ref[...],
            dimension_numbers=(((0,), (0,)), ((), ())),
            preferred_element_type=jnp.float32) + bp_ref[...]


def kernel(x, eigvecs, eigvals, eig_mask, W1, b1, W2, b2, Wp, bp):
    evt = eigvecs.T
    evals_col = eigvals.reshape(K, 1)
    mask_col = eig_mask.astype(jnp.float32).reshape(K, 1)
    w1t = W1.T
    b1_col = b1.reshape(K // 2, 1)
    w2t = W2.T
    b2_col = b2.reshape(K, 1)
    bp_row = bp.reshape(1, OUT)

    out = pl.pallas_call(
        _body,
        grid=(2, NT),
        in_specs=[
            pl.BlockSpec((TN, D), lambda p, j: ((1 - p) * j + p * (NT - 1), 0)),
            pl.BlockSpec((K, N), lambda p, j: (0, 0)),
            pl.BlockSpec((K, 1), lambda p, j: (0, 0)),
            pl.BlockSpec((K, 1), lambda p, j: (0, 0)),
            pl.BlockSpec((K // 2, K), lambda p, j: (0, 0)),
            pl.BlockSpec((K // 2, 1), lambda p, j: (0, 0)),
            pl.BlockSpec((K, K // 2), lambda p, j: (0, 0)),
            pl.BlockSpec((K, 1), lambda p, j: (0, 0)),
            pl.BlockSpec((D, OUT), lambda p, j: (0, 0)),
            pl.BlockSpec((1, OUT), lambda p, j: (0, 0)),
        ],
        out_specs=pl.BlockSpec((TN, OUT), lambda p, j: (p * j, 0)),
        out_shape=jax.ShapeDtypeStruct((N, OUT), jnp.float32),
        scratch_shapes=[pltpu.VMEM((K, D), jnp.float32),
                        pltpu.VMEM((K, OUT), jnp.float32)],
    )(x, evt, evals_col, mask_col, w1t, b1_col, w2t, b2_col, Wp, bp_row)
    return out


# speedup vs baseline: 3.7071x; 1.0453x over previous
import jax
import jax.numpy as jnp
from jax.experimental import pallas as pl
from jax.experimental.pallas import tpu as pltpu

N = 100000
D = 128
K = 16
OUT = 256
TN = 12800
NT = (N + TN - 1) // TN
TAIL = N - (NT - 1) * TN


def _body(x_ref, evt_ref, evals_ref, mask_ref, w1t_ref, b1_ref, w2t_ref,
          b2_ref, wp_ref, bp_ref, out_ref, acc_ref, m_ref):
    p = pl.program_id(0)
    j = pl.program_id(1)

    @pl.when(jnp.logical_and(p == 0, j == 0))
    def _():
        acc_ref[...] = jnp.zeros_like(acc_ref)

    @pl.when(jnp.logical_and(p == 0, j < NT - 1))
    def _():
        evt = evt_ref[:, pl.ds(j * TN, TN)]
        acc_ref[...] += jax.lax.dot_general(
            evt, x_ref[...],
            dimension_numbers=(((1,), (0,)), ((), ())),
            preferred_element_type=jnp.float32)

    @pl.when(jnp.logical_and(p == 0, j == NT - 1))
    def _():
        evt = evt_ref[:, pl.ds((NT - 1) * TN, TAIL)]
        acc_ref[...] += jax.lax.dot_general(
            evt, x_ref[0:TAIL, :],
            dimension_numbers=(((1,), (0,)), ((), ())),
            preferred_element_type=jnp.float32)

    @pl.when(jnp.logical_and(p == 1, j == 0))
    def _():
        h = jnp.maximum(
            jnp.dot(w1t_ref[...], evals_ref[...],
                    preferred_element_type=jnp.float32) + b1_ref[...], 0.0)
        filt = jnp.tanh(
            jnp.dot(w2t_ref[...], h,
                    preferred_element_type=jnp.float32) + b2_ref[...])
        filt = filt * mask_ref[...]
        m_ref[...] = jnp.dot(filt * acc_ref[...], wp_ref[...],
                             preferred_element_type=jnp.float32)

    @pl.when(jnp.logical_and(p == 1, j < NT - 1))
    def _():
        evt = evt_ref[:, pl.ds(j * TN, TN)]
        out_ref[...] = jax.lax.dot_general(
            evt, m_ref[...],
            dimension_numbers=(((0,), (0,)), ((), ())),
            preferred_element_type=jnp.float32) + bp_ref[...]

    @pl.when(jnp.logical_and(p == 1, j == NT - 1))
    def _():
        evt = evt_ref[:, pl.ds((NT - 1) * TN, TAIL)]
        out_ref[0:TAIL, :] = jax.lax.dot_general(
            evt, m_ref[...],
            dimension_numbers=(((0,), (0,)), ((), ())),
            preferred_element_type=jnp.float32) + bp_ref[...]


def kernel(x, eigvecs, eigvals, eig_mask, W1, b1, W2, b2, Wp, bp):
    evt = eigvecs.T
    evals_col = eigvals.reshape(K, 1)
    mask_col = eig_mask.astype(jnp.float32).reshape(K, 1)
    w1t = W1.T
    b1_col = b1.reshape(K // 2, 1)
    w2t = W2.T
    b2_col = b2.reshape(K, 1)
    bp_row = bp.reshape(1, OUT)

    out = pl.pallas_call(
        _body,
        grid=(2, NT),
        in_specs=[
            pl.BlockSpec((TN, D), lambda p, j: ((1 - p) * j + p * (NT - 1), 0)),
            pl.BlockSpec((K, N), lambda p, j: (0, 0)),
            pl.BlockSpec((K, 1), lambda p, j: (0, 0)),
            pl.BlockSpec((K, 1), lambda p, j: (0, 0)),
            pl.BlockSpec((K // 2, K), lambda p, j: (0, 0)),
            pl.BlockSpec((K // 2, 1), lambda p, j: (0, 0)),
            pl.BlockSpec((K, K // 2), lambda p, j: (0, 0)),
            pl.BlockSpec((K, 1), lambda p, j: (0, 0)),
            pl.BlockSpec((D, OUT), lambda p, j: (0, 0)),
            pl.BlockSpec((1, OUT), lambda p, j: (0, 0)),
        ],
        out_specs=pl.BlockSpec((TN, OUT), lambda p, j: (p * j, 0)),
        out_shape=jax.ShapeDtypeStruct((N, OUT), jnp.float32),
        scratch_shapes=[pltpu.VMEM((K, D), jnp.float32),
                        pltpu.VMEM((K, OUT), jnp.float32)],
    )(x, evt, evals_col, mask_col, w1t, b1_col, w2t, b2_col, Wp, bp_row)
    return out
